# Initial kernel scaffold; baseline (speedup 1.0000x reference)
import jax
import jax.numpy as jnp
from jax.experimental import pallas as pl


def kernel(kp1, w_kp1, kp1_desc, desc2, homo12):
    raise NotImplementedError("write your pallas kernel here")


# speedup vs baseline: 37.4248x; 37.4248x over previous
import jax
import jax.numpy as jnp
import numpy as np
from jax.experimental import pallas as pl
from jax.experimental.pallas import tpu as pltpu

_GS = 8.0
_NNEG = 16
_SNEG = 8
_B, _N, _C, _H, _W = 4, 512, 128, 64, 64
_HW = _H * _W
_RCH = 128
_NCH = _N // _RCH
_BIGF = 3.0e38
_BIGID = 1.0e9
_RADIUS = np.float32(_GS * np.sqrt(2.0) + 0.1)


def _mmt(a, b):
    return jax.lax.dot_general(
        a, b, (((1,), (1,)), ((), ())),
        preferred_element_type=jnp.float32,
        precision=jax.lax.Precision.HIGHEST)


def _mm(a, b):
    return jax.lax.dot_general(
        a, b, (((1,), (0,)), ((), ())),
        preferred_element_type=jnp.float32,
        precision=jax.lax.Precision.HIGHEST)


def _nearest4(x, y):
    u = jnp.clip(x * (1.0 / _GS) - 0.5, -0.5, 63.5)
    v = jnp.clip(y * (1.0 / _GS) - 0.5, -0.5, 63.5)
    jx = jnp.floor(u)
    jy = jnp.floor(v)
    dists = []
    ids = []
    for dr in (-1.0, 0.0, 1.0, 2.0):
        for dc in (-1.0, 0.0, 1.0, 2.0):
            c = jx + dc
            r = jy + dr
            valid = (c >= 0.0) & (c <= 63.0) & (r >= 0.0) & (r <= 63.0)
            gx = (c + 0.5) * _GS
            gy = (r + 0.5) * _GS
            dx = x - gx
            dy = y - gy
            d = jnp.sqrt(dx * dx + dy * dy + 1e-12)
            dists.append(jnp.where(valid, d, _BIGF))
            ids.append(jnp.where(valid, r * 64.0 + c, _BIGID))
    sels = []
    for _ in range(4):
        m = dists[0]
        for d in dists[1:]:
            m = jnp.minimum(m, d)
        t = [jnp.where(d == m, i, _BIGID) for d, i in zip(dists, ids)]
        sel = t[0]
        for tt in t[1:]:
            sel = jnp.minimum(sel, tt)
        dists = [jnp.where(i == sel, _BIGF, d) for d, i in zip(dists, ids)]
        sels.append(sel)
    return sels


def _k1_body(kp1_ref, wkp1_ref, desc2_ref, wdesc_ref, cells_ref):
    x = kp1_ref[0, :, 0:1]
    y = kp1_ref[0, :, 1:2]
    sels = _nearest4(x, y)
    for k in range(4):
        cells_ref[0, :, k:k + 1] = sels[k].astype(jnp.int32)
    cx = wkp1_ref[0, :, 0:1] * (1.0 / _GS) - 0.5
    cy = wkp1_ref[0, :, 1:2] * (1.0 / _GS) - 0.5
    x0 = jnp.floor(cx)
    y0 = jnp.floor(cy)
    wx1 = cx - x0
    wx0 = 1.0 - wx1
    wy1 = cy - y0
    wy0 = 1.0 - wy1
    x0c = jnp.clip(x0, 0.0, _W - 1.0)
    x1c = jnp.clip(x0 + 1.0, 0.0, _W - 1.0)
    y0c = jnp.clip(y0, 0.0, _H - 1.0)
    y1c = jnp.clip(y0 + 1.0, 0.0, _H - 1.0)
    iot = jax.lax.broadcasted_iota(jnp.int32, (_RCH, _HW), 1).astype(jnp.float32)
    wmat = (jnp.where(iot == y0c * 64.0 + x0c, wy0 * wx0, 0.0)
            + jnp.where(iot == y0c * 64.0 + x1c, wy0 * wx1, 0.0)
            + jnp.where(iot == y1c * 64.0 + x0c, wy1 * wx0, 0.0)
            + jnp.where(iot == y1c * 64.0 + x1c, wy1 * wx1, 0.0))
    raw = _mm(wmat, desc2_ref[0])
    inv = 1.0 / jnp.sqrt(jnp.sum(raw * raw, axis=1, keepdims=True) + 1e-8)
    wdesc_ref[0] = raw * inv


def _k2_body(kdesc_ref, desc2_ref, wdesc_ref, wcx_ref, wcy_ref, out_ref):
    wx = wcx_ref[0]
    wy = wcy_ref[0]
    sels = _nearest4(wx, wy)
    iot = jax.lax.broadcasted_iota(jnp.int32, (_RCH, _HW), 1).astype(jnp.float32)
    mask = jnp.zeros((_RCH, _HW), jnp.float32)
    for k in range(4):
        for g in range(4):
            mask = mask + jnp.where(iot == sels[k][:, g:g + 1], 1.0, 0.0)
    kdesc = kdesc_ref[0]
    sim = 2.0 - 2.0 * _mmt(kdesc, desc2_ref[0]) + mask * 5.0
    pos = 2.0 - 2.0 * jnp.sum(kdesc * wdesc_ref[0], axis=1, keepdims=True)
    rem = jnp.full((_RCH, 1), 16.0, jnp.float32)
    total = jnp.zeros((_RCH, 1), jnp.float32)
    for _ in range(_NNEG):
        m = jnp.min(sim, axis=1, keepdims=True)
        eq = sim == m
        cnt = jnp.sum(eq.astype(jnp.float32), axis=1, keepdims=True)
        w = jnp.minimum(cnt, rem)
        term = jnp.maximum(pos - m + 1.0, 0.0) ** 2
        total = total + w * term
        rem = rem - w
        sim = jnp.where(eq, _BIGF, sim)
    out_ref[0, 0, 0, 0] = jnp.sum(total)


def _top8_dots(sim, gram):
    n = sim.shape[1]
    iot = jax.lax.broadcasted_iota(jnp.int32, sim.shape, 1).astype(jnp.float32)
    dots = []
    for _ in range(_SNEG):
        m = jnp.min(sim, axis=1, keepdims=True)
        t = jnp.where(sim == m, iot, _BIGID)
        j = jnp.min(t, axis=1, keepdims=True)
        hit = iot == j
        dots.append(jnp.sum(jnp.where(hit, gram, 0.0), axis=1, keepdims=True))
        sim = jnp.where(hit, _BIGF, sim)
    return dots


def _k3_body(kp1_ref, kp1t_ref, wkp1_ref, wkp1t_ref, kdesc_ref, wdesc_ref,
             out_ref):
    kdesc = kdesc_ref[0]
    wdesc = wdesc_ref[0]
    g1 = _mmt(kdesc, kdesc)
    g2 = _mmt(wdesc, wdesc)
    x = kp1_ref[0, :, 0:1]
    y = kp1_ref[0, :, 1:2]
    xr = kp1t_ref[0, 0:1, :]
    yr = kp1t_ref[0, 1:2, :]
    dx = x - xr
    dy = y - yr
    d1 = jnp.sqrt(dx * dx + dy * dy + 1e-12)
    sim1 = 2.0 - 2.0 * g1 + jnp.where(d1 <= _RADIUS, 5.0, 0.0)
    wx = wkp1_ref[0, :, 0:1]
    wy = wkp1_ref[0, :, 1:2]
    wxr = wkp1t_ref[0, 0:1, :]
    wyr = wkp1t_ref[0, 1:2, :]
    dwx = wx - wxr
    dwy = wy - wyr
    d2 = jnp.sqrt(dwx * dwx + dwy * dwy + 1e-12)
    sim2 = 2.0 - 2.0 * g2 + jnp.where(d2 <= _RADIUS, 5.0, 0.0)
    a = _top8_dots(sim1, g1)
    b = _top8_dots(sim2, g2)
    acc = jnp.zeros((_N, 1), jnp.float32)
    for k in range(_SNEG):
        s = (2.0 - 2.0 * a[k]) - (2.0 - 2.0 * b[k])
        acc = acc + s * s
    out_ref[0, 0, 0] = jnp.sum(jnp.sqrt(acc + 1e-12))


def kernel(kp1, w_kp1, kp1_desc, desc2, homo12):
    b, n, c = kp1_desc.shape
    desc2_flat = jnp.transpose(desc2, (0, 2, 3, 1)).reshape(b, _HW, c)

    wdesc, cells = pl.pallas_call(
        _k1_body,
        grid=(_B, _NCH),
        in_specs=[
            pl.BlockSpec((1, _RCH, 2), lambda bi, i: (bi, i, 0)),
            pl.BlockSpec((1, _RCH, 2), lambda bi, i: (bi, i, 0)),
            pl.BlockSpec((1, _HW, _C), lambda bi, i: (bi, 0, 0)),
        ],
        out_specs=[
            pl.BlockSpec((1, _RCH, _C), lambda bi, i: (bi, i, 0)),
            pl.BlockSpec((1, _RCH, 4), lambda bi, i: (bi, i, 0)),
        ],
        out_shape=[
            jax.ShapeDtypeStruct((_B, _N, _C), jnp.float32),
            jax.ShapeDtypeStruct((_B, _N, 4), jnp.int32),
        ],
    )(kp1, w_kp1, desc2_flat)

    cc = jnp.remainder(cells, 64).astype(jnp.float32)
    rr = (cells // 64).astype(jnp.float32)
    px = (cc + 0.5) * _GS
    py = (rr + 0.5) * _GS
    pts = jnp.stack([px, py], axis=-1).reshape(b, n * 4, 2)
    ones = jnp.ones(pts.shape[:-1] + (1,), pts.dtype)
    ph = jnp.concatenate([pts, ones], axis=-1)
    wp = jnp.einsum('bij,bnj->bni', homo12, ph)
    wpts = wp[..., :2] / (wp[..., 2:3] + 1e-8)
    wcx = wpts[..., 0].reshape(b, n, 4)
    wcy = wpts[..., 1].reshape(b, n, 4)

    fos_parts = pl.pallas_call(
        _k2_body,
        grid=(_B, _NCH),
        in_specs=[
            pl.BlockSpec((1, _RCH, _C), lambda bi, i: (bi, i, 0)),
            pl.BlockSpec((1, _HW, _C), lambda bi, i: (bi, 0, 0)),
            pl.BlockSpec((1, _RCH, _C), lambda bi, i: (bi, i, 0)),
            pl.BlockSpec((1, _RCH, 4), lambda bi, i: (bi, i, 0)),
            pl.BlockSpec((1, _RCH, 4), lambda bi, i: (bi, i, 0)),
        ],
        out_specs=pl.BlockSpec((1, 1, 1, 1), lambda bi, i: (bi, i, 0, 0),
                               memory_space=pltpu.SMEM),
        out_shape=jax.ShapeDtypeStruct((_B, _NCH, 1, 1), jnp.float32),
    )(kp1_desc, desc2_flat, wdesc, wcx, wcy)

    kp1_t = jnp.transpose(kp1, (0, 2, 1))
    wkp1_t = jnp.transpose(w_kp1, (0, 2, 1))
    sos_parts = pl.pallas_call(
        _k3_body,
        grid=(_B,),
        in_specs=[
            pl.BlockSpec((1, _N, 2), lambda bi: (bi, 0, 0)),
            pl.BlockSpec((1, 2, _N), lambda bi: (bi, 0, 0)),
            pl.BlockSpec((1, _N, 2), lambda bi: (bi, 0, 0)),
            pl.BlockSpec((1, 2, _N), lambda bi: (bi, 0, 0)),
            pl.BlockSpec((1, _N, _C), lambda bi: (bi, 0, 0)),
            pl.BlockSpec((1, _N, _C), lambda bi: (bi, 0, 0)),
        ],
        out_specs=pl.BlockSpec((1, 1, 1), lambda bi: (bi, 0, 0),
                               memory_space=pltpu.SMEM),
        out_shape=jax.ShapeDtypeStruct((_B, 1, 1), jnp.float32),
    )(kp1, kp1_t, w_kp1, wkp1_t, kp1_desc, wdesc)

    fos = jnp.sum(fos_parts) / (b * n * _NNEG)
    sos = jnp.sum(sos_parts) / (b * n)
    return fos + sos


# speedup vs baseline: 40.7805x; 1.0897x over previous
import jax
import jax.numpy as jnp
import numpy as np
from jax.experimental import pallas as pl
from jax.experimental.pallas import tpu as pltpu

_GS = 8.0
_NNEG = 16
_SNEG = 8
_B, _N, _C, _H, _W = 4, 512, 128, 64, 64
_HW = _H * _W
_RCH = 128
_NCH = _N // _RCH
_BIGF = 3.0e38
_BIGID = 1.0e9
_RADIUS = np.float32(_GS * np.sqrt(2.0) + 0.1)


def _mmt(a, b):
    return jax.lax.dot_general(
        a, b, (((1,), (1,)), ((), ())),
        preferred_element_type=jnp.float32,
        precision=jax.lax.Precision.DEFAULT)


def _mm(a, b):
    return jax.lax.dot_general(
        a, b, (((1,), (0,)), ((), ())),
        preferred_element_type=jnp.float32,
        precision=jax.lax.Precision.DEFAULT)


def _nearest4(x, y):
    u = jnp.clip(x * (1.0 / _GS) - 0.5, -0.5, 63.5)
    v = jnp.clip(y * (1.0 / _GS) - 0.5, -0.5, 63.5)
    jx = jnp.floor(u)
    jy = jnp.floor(v)
    dists = []
    ids = []
    for dr in (-1.0, 0.0, 1.0, 2.0):
        for dc in (-1.0, 0.0, 1.0, 2.0):
            c = jx + dc
            r = jy + dr
            valid = (c >= 0.0) & (c <= 63.0) & (r >= 0.0) & (r <= 63.0)
            gx = (c + 0.5) * _GS
            gy = (r + 0.5) * _GS
            dx = x - gx
            dy = y - gy
            d = jnp.sqrt(dx * dx + dy * dy + 1e-12)
            dists.append(jnp.where(valid, d, _BIGF))
            ids.append(jnp.where(valid, r * 64.0 + c, _BIGID))
    sels = []
    for _ in range(4):
        m = dists[0]
        for d in dists[1:]:
            m = jnp.minimum(m, d)
        t = [jnp.where(d == m, i, _BIGID) for d, i in zip(dists, ids)]
        sel = t[0]
        for tt in t[1:]:
            sel = jnp.minimum(sel, tt)
        dists = [jnp.where(i == sel, _BIGF, d) for d, i in zip(dists, ids)]
        sels.append(sel)
    return sels


def _k1_body(kp1_ref, wkp1_ref, desc2_ref, wdesc_ref, cells_ref):
    x = kp1_ref[0, :, 0:1]
    y = kp1_ref[0, :, 1:2]
    sels = _nearest4(x, y)
    for k in range(4):
        cells_ref[0, :, k:k + 1] = sels[k].astype(jnp.int32)
    cx = wkp1_ref[0, :, 0:1] * (1.0 / _GS) - 0.5
    cy = wkp1_ref[0, :, 1:2] * (1.0 / _GS) - 0.5
    x0 = jnp.floor(cx)
    y0 = jnp.floor(cy)
    wx1 = cx - x0
    wx0 = 1.0 - wx1
    wy1 = cy - y0
    wy0 = 1.0 - wy1
    x0c = jnp.clip(x0, 0.0, _W - 1.0)
    x1c = jnp.clip(x0 + 1.0, 0.0, _W - 1.0)
    y0c = jnp.clip(y0, 0.0, _H - 1.0)
    y1c = jnp.clip(y0 + 1.0, 0.0, _H - 1.0)
    iot = jax.lax.broadcasted_iota(jnp.int32, (_RCH, _HW), 1).astype(jnp.float32)
    wmat = (jnp.where(iot == y0c * 64.0 + x0c, wy0 * wx0, 0.0)
            + jnp.where(iot == y0c * 64.0 + x1c, wy0 * wx1, 0.0)
            + jnp.where(iot == y1c * 64.0 + x0c, wy1 * wx0, 0.0)
            + jnp.where(iot == y1c * 64.0 + x1c, wy1 * wx1, 0.0))
    raw = _mm(wmat, desc2_ref[0])
    inv = 1.0 / jnp.sqrt(jnp.sum(raw * raw, axis=1, keepdims=True) + 1e-8)
    wdesc_ref[0] = raw * inv


def _k2_body(kdesc_ref, desc2_ref, wdesc_ref, wcx_ref, wcy_ref, out_ref):
    wx = wcx_ref[0]
    wy = wcy_ref[0]
    sels = _nearest4(wx, wy)
    iot = jax.lax.broadcasted_iota(jnp.int32, (_RCH, _HW), 1).astype(jnp.float32)
    excl = iot == sels[0][:, 0:1]
    for k in range(4):
        for g in range(4):
            if k == 0 and g == 0:
                continue
            excl = excl | (iot == sels[k][:, g:g + 1])
    kdesc = kdesc_ref[0]
    sim = 2.0 - 2.0 * _mmt(kdesc, desc2_ref[0])
    sim = jnp.where(excl, _BIGF, sim)
    pos = 2.0 - 2.0 * jnp.sum(kdesc * wdesc_ref[0], axis=1, keepdims=True)
    rem = jnp.full((_RCH, 1), 16.0, jnp.float32)
    total = jnp.zeros((_RCH, 1), jnp.float32)
    for _ in range(_NNEG):
        m = jnp.min(sim, axis=1, keepdims=True)
        eq = sim == m
        cnt = jnp.sum(eq.astype(jnp.float32), axis=1, keepdims=True)
        w = jnp.minimum(cnt, rem)
        term = jnp.maximum(pos - m + 1.0, 0.0) ** 2
        total = total + w * term
        rem = rem - w
        sim = jnp.where(eq, _BIGF, sim)
    out_ref[0, 0, 0, 0] = jnp.sum(total)


def _top8_dots(sim, gram):
    n = sim.shape[1]
    iot = jax.lax.broadcasted_iota(jnp.int32, sim.shape, 1).astype(jnp.float32)
    dots = []
    for _ in range(_SNEG):
        m = jnp.min(sim, axis=1, keepdims=True)
        t = jnp.where(sim == m, iot, _BIGID)
        j = jnp.min(t, axis=1, keepdims=True)
        hit = iot == j
        dots.append(jnp.sum(jnp.where(hit, gram, 0.0), axis=1, keepdims=True))
        sim = jnp.where(hit, _BIGF, sim)
    return dots


def _k3_body(kp1_ref, kp1t_ref, wkp1_ref, wkp1t_ref, kdesc_ref, wdesc_ref,
             out_ref):
    kdesc = kdesc_ref[0]
    wdesc = wdesc_ref[0]
    g1 = _mmt(kdesc, kdesc)
    g2 = _mmt(wdesc, wdesc)
    x = kp1_ref[0, :, 0:1]
    y = kp1_ref[0, :, 1:2]
    xr = kp1t_ref[0, 0:1, :]
    yr = kp1t_ref[0, 1:2, :]
    dx = x - xr
    dy = y - yr
    d1 = jnp.sqrt(dx * dx + dy * dy + 1e-12)
    sim1 = 2.0 - 2.0 * g1 + jnp.where(d1 <= _RADIUS, 5.0, 0.0)
    wx = wkp1_ref[0, :, 0:1]
    wy = wkp1_ref[0, :, 1:2]
    wxr = wkp1t_ref[0, 0:1, :]
    wyr = wkp1t_ref[0, 1:2, :]
    dwx = wx - wxr
    dwy = wy - wyr
    d2 = jnp.sqrt(dwx * dwx + dwy * dwy + 1e-12)
    sim2 = 2.0 - 2.0 * g2 + jnp.where(d2 <= _RADIUS, 5.0, 0.0)
    a = _top8_dots(sim1, g1)
    b = _top8_dots(sim2, g2)
    acc = jnp.zeros((_N, 1), jnp.float32)
    for k in range(_SNEG):
        s = (2.0 - 2.0 * a[k]) - (2.0 - 2.0 * b[k])
        acc = acc + s * s
    out_ref[0, 0, 0] = jnp.sum(jnp.sqrt(acc + 1e-12))


def kernel(kp1, w_kp1, kp1_desc, desc2, homo12):
    b, n, c = kp1_desc.shape
    desc2_flat = jnp.transpose(desc2, (0, 2, 3, 1)).reshape(b, _HW, c)

    wdesc, cells = pl.pallas_call(
        _k1_body,
        grid=(_B, _NCH),
        in_specs=[
            pl.BlockSpec((1, _RCH, 2), lambda bi, i: (bi, i, 0)),
            pl.BlockSpec((1, _RCH, 2), lambda bi, i: (bi, i, 0)),
            pl.BlockSpec((1, _HW, _C), lambda bi, i: (bi, 0, 0)),
        ],
        out_specs=[
            pl.BlockSpec((1, _RCH, _C), lambda bi, i: (bi, i, 0)),
            pl.BlockSpec((1, _RCH, 4), lambda bi, i: (bi, i, 0)),
        ],
        out_shape=[
            jax.ShapeDtypeStruct((_B, _N, _C), jnp.float32),
            jax.ShapeDtypeStruct((_B, _N, 4), jnp.int32),
        ],
    )(kp1, w_kp1, desc2_flat)

    cc = jnp.remainder(cells, 64).astype(jnp.float32)
    rr = (cells // 64).astype(jnp.float32)
    px = (cc + 0.5) * _GS
    py = (rr + 0.5) * _GS
    pts = jnp.stack([px, py], axis=-1).reshape(b, n * 4, 2)
    ones = jnp.ones(pts.shape[:-1] + (1,), pts.dtype)
    ph = jnp.concatenate([pts, ones], axis=-1)
    wp = jnp.einsum('bij,bnj->bni', homo12, ph)
    wpts = wp[..., :2] / (wp[..., 2:3] + 1e-8)
    wcx = wpts[..., 0].reshape(b, n, 4)
    wcy = wpts[..., 1].reshape(b, n, 4)

    fos_parts = pl.pallas_call(
        _k2_body,
        grid=(_B, _NCH),
        in_specs=[
            pl.BlockSpec((1, _RCH, _C), lambda bi, i: (bi, i, 0)),
            pl.BlockSpec((1, _HW, _C), lambda bi, i: (bi, 0, 0)),
            pl.BlockSpec((1, _RCH, _C), lambda bi, i: (bi, i, 0)),
            pl.BlockSpec((1, _RCH, 4), lambda bi, i: (bi, i, 0)),
            pl.BlockSpec((1, _RCH, 4), lambda bi, i: (bi, i, 0)),
        ],
        out_specs=pl.BlockSpec((1, 1, 1, 1), lambda bi, i: (bi, i, 0, 0),
                               memory_space=pltpu.SMEM),
        out_shape=jax.ShapeDtypeStruct((_B, _NCH, 1, 1), jnp.float32),
    )(kp1_desc, desc2_flat, wdesc, wcx, wcy)

    kp1_t = jnp.transpose(kp1, (0, 2, 1))
    wkp1_t = jnp.transpose(w_kp1, (0, 2, 1))
    sos_parts = pl.pallas_call(
        _k3_body,
        grid=(_B,),
        in_specs=[
            pl.BlockSpec((1, _N, 2), lambda bi: (bi, 0, 0)),
            pl.BlockSpec((1, 2, _N), lambda bi: (bi, 0, 0)),
            pl.BlockSpec((1, _N, 2), lambda bi: (bi, 0, 0)),
            pl.BlockSpec((1, 2, _N), lambda bi: (bi, 0, 0)),
            pl.BlockSpec((1, _N, _C), lambda bi: (bi, 0, 0)),
            pl.BlockSpec((1, _N, _C), lambda bi: (bi, 0, 0)),
        ],
        out_specs=pl.BlockSpec((1, 1, 1), lambda bi: (bi, 0, 0),
                               memory_space=pltpu.SMEM),
        out_shape=jax.ShapeDtypeStruct((_B, 1, 1), jnp.float32),
    )(kp1, kp1_t, w_kp1, wkp1_t, kp1_desc, wdesc)

    fos = jnp.sum(fos_parts) / (b * n * _NNEG)
    sos = jnp.sum(sos_parts) / (b * n)
    return fos + sos
